# baseline (speedup 1.0000x reference)
import functools

import jax
import jax.numpy as jnp
from jax import lax
from jax.experimental import pallas as pl
from jax.experimental.pallas import tpu as pltpu
from jax.experimental.pallas import tpu_sc as plsc

B, D, C = 16384, 128, 4
NC, NS, L = 2, 16, 16
NW = NC * NS
B_PER_W = B // NW
CHUNK = 128
N_CHUNKS = B_PER_W // CHUNK
VPR = D // L
NSLOT = 2

_mesh = plsc.VectorSubcoreMesh(core_axis_name="c", subcore_axis_name="s")


@functools.partial(
    pl.kernel,
    out_type=(
        jax.ShapeDtypeStruct((B, D), jnp.float32),
        jax.ShapeDtypeStruct((B, D), jnp.float32),
    ),
    mesh=_mesh,
    scratch_types=[
        pltpu.VMEM((B_PER_W + L,), jnp.int32),
        pltpu.VMEM((C + 1, D), jnp.float32),
        pltpu.VMEM((C + 1, D), jnp.float32),
        pltpu.VMEM((NSLOT, CHUNK, D), jnp.float32),
        pltpu.VMEM((NSLOT, CHUNK, D), jnp.float32),
        pltpu.SemaphoreType.DMA,
        pltpu.SemaphoreType.DMA,
    ],
)
def _sc_lookup(latent_hbm, mu_ext_hbm, ls_ext_hbm,
               mu_out, ls_out,
               idx_all, mu_tab, ls_tab, mu_v, ls_v,
               sem_out0, sem_out1):
    wid = lax.axis_index("s") * NC + lax.axis_index("c")
    base = wid * B_PER_W
    sem_out = (sem_out0, sem_out1)

    pltpu.sync_copy(latent_hbm.at[pl.ds(base, B_PER_W)],
                    idx_all.at[pl.ds(0, B_PER_W)])
    pltpu.sync_copy(mu_ext_hbm, mu_tab)
    pltpu.sync_copy(ls_ext_hbm, ls_tab)

    def outs(s, ci):
        off = base + ci * CHUNK
        dst = pl.ds(off, CHUNK)
        return (
            pltpu.make_async_copy(mu_v.at[s], mu_out.at[dst], sem_out[s]),
            pltpu.make_async_copy(ls_v.at[s], ls_out.at[dst], sem_out[s]),
        )

    def compute(s, ci):
        def row_body(r, _):
            v = idx_all[pl.ds(ci * CHUNK + r, L)][0]
            c = jnp.where(v < 0, C, jnp.minimum(jnp.maximum(v, 0), C - 1))
            for j in range(VPR):
                sl = pl.ds(j * L, L)
                mu_v[s, r, sl] = mu_tab[c, sl]
                ls_v[s, r, sl] = ls_tab[c, sl]
            return 0

        lax.fori_loop(0, CHUNK, row_body, 0, unroll=8)

    for ci in range(N_CHUNKS):
        s = ci % NSLOT
        if ci >= NSLOT:
            for cp in outs(s, ci - NSLOT):
                cp.wait()
        compute(s, ci)
        for cp in outs(s, ci):
            cp.start()
    for ci in (N_CHUNKS - 2, N_CHUNKS - 1):
        for cp in outs(ci % NSLOT, ci):
            cp.wait()


BR = 2048


def _tc_sample(lat_ref, noise_ref, mu_ref, ls_ref, out_ref):
    lat = lat_ref[...]
    c = jnp.where(lat < 0, C, jnp.clip(lat, 0, C - 1))
    oh = (c == lax.broadcasted_iota(jnp.int32, (BR, C + 1), 1))
    oh = oh.astype(jnp.float32)
    mu = jnp.dot(oh, mu_ref[...], preferred_element_type=jnp.float32)
    sig = jnp.exp(jnp.dot(oh, ls_ref[...],
                          preferred_element_type=jnp.float32))
    out_ref[...] = mu + noise_ref[...] * sig


_tc_call = pl.pallas_call(
    _tc_sample,
    grid=(B // BR,),
    in_specs=[
        pl.BlockSpec((BR, 1), lambda i: (i, 0)),
        pl.BlockSpec((BR, D), lambda i: (i, 0)),
        pl.BlockSpec((C + 1, D), lambda i: (0, 0)),
        pl.BlockSpec((C + 1, D), lambda i: (0, 0)),
    ],
    out_specs=pl.BlockSpec((BR, D), lambda i: (i, 0)),
    out_shape=jax.ShapeDtypeStruct((B, D), jnp.float32),
)


def kernel(latent, noise, mu_table, log_sigma_table, online_mu,
           online_log_sigma):
    mu_ext = jnp.concatenate([mu_table, online_mu[None, :]], axis=0)
    ls_ext = jnp.concatenate([log_sigma_table, online_log_sigma[None, :]],
                             axis=0)
    latent = latent.astype(jnp.int32)
    sample = _tc_call(latent[:, None], noise, mu_ext, ls_ext)
    mu, ls = _sc_lookup(latent, mu_ext, ls_ext)
    return (mu, ls, sample)


# speedup vs baseline: 1.3544x; 1.3544x over previous
import functools

import jax
import jax.numpy as jnp
from jax import lax
from jax.experimental import pallas as pl
from jax.experimental.pallas import tpu as pltpu
from jax.experimental.pallas import tpu_sc as plsc

B, D, C = 16384, 128, 4
NC, NS, L = 2, 16, 16
NW = NC * NS
B_PER_W = B // NW
CHUNK = 128
N_CHUNKS = B_PER_W // CHUNK
VPR = D // L
NSLOT = 2

_mesh = plsc.VectorSubcoreMesh(core_axis_name="c", subcore_axis_name="s")


@functools.partial(
    pl.kernel,
    out_type=jax.ShapeDtypeStruct((B, D), jnp.float32),
    mesh=_mesh,
    scratch_types=[
        pltpu.VMEM((B_PER_W + L,), jnp.int32),
        pltpu.VMEM((C + 1, D), jnp.float32),
        pltpu.VMEM((NSLOT, CHUNK, D), jnp.float32),
        pltpu.SemaphoreType.DMA,
        pltpu.SemaphoreType.DMA,
    ],
)
def _sc_lookup(latent_hbm, ls_ext_hbm, ls_out,
               idx_all, ls_tab, ls_v, sem_out0, sem_out1):
    wid = lax.axis_index("s") * NC + lax.axis_index("c")
    base = wid * B_PER_W
    sem_out = (sem_out0, sem_out1)

    pltpu.sync_copy(latent_hbm.at[pl.ds(base, B_PER_W)],
                    idx_all.at[pl.ds(0, B_PER_W)])
    pltpu.sync_copy(ls_ext_hbm, ls_tab)

    def out_cp(s, ci):
        off = base + ci * CHUNK
        return pltpu.make_async_copy(ls_v.at[s], ls_out.at[pl.ds(off, CHUNK)],
                                     sem_out[s])

    def compute(s, ci):
        def row_body(r, _):
            v = idx_all[pl.ds(ci * CHUNK + r, L)][0]
            c = jnp.where(v < 0, C, jnp.minimum(jnp.maximum(v, 0), C - 1))
            for j in range(VPR):
                sl = pl.ds(j * L, L)
                ls_v[s, r, sl] = ls_tab[c, sl]
            return 0

        lax.fori_loop(0, CHUNK, row_body, 0, unroll=8)

    for ci in range(N_CHUNKS):
        s = ci % NSLOT
        if ci >= NSLOT:
            out_cp(s, ci - NSLOT).wait()
        compute(s, ci)
        out_cp(s, ci).start()
    for ci in (N_CHUNKS - 2, N_CHUNKS - 1):
        out_cp(ci % NSLOT, ci).wait()


BR = 2048


def _tc_sample(lat_ref, noise_ref, mu_ref, ls_ref, mu_out_ref, samp_ref):
    lat = lat_ref[...]
    c = jnp.where(lat < 0, C, jnp.clip(lat, 0, C - 1))
    oh = (c == lax.broadcasted_iota(jnp.int32, (BR, C + 1), 1))
    oh = oh.astype(jnp.float32)
    mu = jnp.dot(oh, mu_ref[...], preferred_element_type=jnp.float32)
    sig = jnp.exp(jnp.dot(oh, ls_ref[...],
                          preferred_element_type=jnp.float32))
    mu_out_ref[...] = mu
    samp_ref[...] = mu + noise_ref[...] * sig


_tc_call = pl.pallas_call(
    _tc_sample,
    grid=(B // BR,),
    in_specs=[
        pl.BlockSpec((BR, 1), lambda i: (i, 0)),
        pl.BlockSpec((BR, D), lambda i: (i, 0)),
        pl.BlockSpec((C + 1, D), lambda i: (0, 0)),
        pl.BlockSpec((C + 1, D), lambda i: (0, 0)),
    ],
    out_specs=[
        pl.BlockSpec((BR, D), lambda i: (i, 0)),
        pl.BlockSpec((BR, D), lambda i: (i, 0)),
    ],
    out_shape=[
        jax.ShapeDtypeStruct((B, D), jnp.float32),
        jax.ShapeDtypeStruct((B, D), jnp.float32),
    ],
)


def kernel(latent, noise, mu_table, log_sigma_table, online_mu,
           online_log_sigma):
    mu_ext = jnp.concatenate([mu_table, online_mu[None, :]], axis=0)
    ls_ext = jnp.concatenate([log_sigma_table, online_log_sigma[None, :]],
                             axis=0)
    latent = latent.astype(jnp.int32)
    ls = _sc_lookup(latent, ls_ext)
    mu, sample = _tc_call(latent[:, None], noise, mu_ext, ls_ext)
    return (mu, ls, sample)
